# Initial kernel scaffold; baseline (speedup 1.0000x reference)
import jax
import jax.numpy as jnp
from jax.experimental import pallas as pl


def kernel(x, edge_index, batch, W0, b0, Ws, bs, gammas, betas, eps, W_out, b_out):
    raise NotImplementedError("write your pallas kernel here")


# speedup vs baseline: 5.6533x; 5.6533x over previous
import functools

import jax
import jax.numpy as jnp
from jax import lax
from jax.experimental import pallas as pl
from jax.experimental.pallas import tpu as pltpu
from jax.experimental.pallas import tpu_sc as plsc

N = 10000
E = 320000
H = 64
G = 128
L = 5

NC = 2
NS = 16
LANES = 128

N_PAD = 10240
ROWS_PER_TILE = N_PAD // NS
EW = -(-E // (NC * NS * LANES)) * LANES
CH = EW // LANES
E_PAD = EW * NC * NS


def _seg_body(h_hbm, src_hbm, dst_hbm, zeros_hbm, out_hbm,
              src_v, dst_v, rows_v, acc_shared, sem):
    c = lax.axis_index("c")
    s = lax.axis_index("s")

    pltpu.sync_copy(src_hbm.at[c, s], src_v)
    pltpu.sync_copy(dst_hbm.at[c, s], dst_v)

    pltpu.sync_copy(zeros_hbm, acc_shared.at[pl.ds(s * ROWS_PER_TILE, ROWS_PER_TILE)])
    plsc.subcore_barrier()

    def body(j, carry):
        pltpu.async_copy(h_hbm.at[src_v.at[j]], rows_v, sem).wait()
        pltpu.sync_copy(rows_v, acc_shared.at[dst_v.at[j]], add=True)
        return carry

    lax.fori_loop(0, CH, body, 0)
    plsc.subcore_barrier()

    sl = pl.ds(s * ROWS_PER_TILE, ROWS_PER_TILE)
    pltpu.sync_copy(acc_shared.at[sl], out_hbm.at[c, sl])


@jax.jit
def _segsum_sc(h, src3, dst3, zeros):
    d = h.shape[1]
    mesh = plsc.VectorSubcoreMesh(core_axis_name="c", subcore_axis_name="s")
    k = pl.kernel(
        _seg_body,
        out_type=jax.ShapeDtypeStruct((NC, N_PAD, d), jnp.float32),
        mesh=mesh,
        scratch_types=[
            pltpu.VMEM((CH, LANES), jnp.int32),
            pltpu.VMEM((CH, LANES), jnp.int32),
            pltpu.VMEM((LANES, d), jnp.float32),
            pltpu.VMEM_SHARED((N_PAD, d), jnp.float32),
            pltpu.SemaphoreType.DMA,
        ],
        compiler_params=pltpu.CompilerParams(use_tc_tiling_on_sc=False),
    )
    return k(h, src3, dst3, zeros)


def _mid_body(e_ref, h_ref, s_ref, w_ref, b_ref, g_ref, beta_ref, o_ref):
    agg = s_ref[0, :N, :] + s_ref[1, :N, :]
    zpre = e_ref[...] * h_ref[...] + agg
    z = jnp.dot(zpre, w_ref[...], preferred_element_type=jnp.float32) + b_ref[...]
    mu = jnp.mean(z, axis=0, keepdims=True)
    d = z - mu
    var = jnp.mean(d * d, axis=0, keepdims=True)
    h = d * lax.rsqrt(var + 1e-5) * g_ref[...] + beta_ref[...]
    o_ref[...] = jnp.maximum(h, 0.0)


@jax.jit
def _mid_tc(e, h, s, w, b, g, beta):
    return pl.pallas_call(
        _mid_body,
        out_shape=jax.ShapeDtypeStruct((N, w.shape[1]), jnp.float32),
    )(e, h, s, w, b, g, beta)


def _last_body(e_ref, h_ref, s_ref, w_ref, b_ref, g_ref, beta_ref, batch_ref,
               wout_ref, bout_ref, o_ref):
    agg = s_ref[0, :N, :] + s_ref[1, :N, :]
    zpre = e_ref[...] * h_ref[...] + agg
    z = jnp.dot(zpre, w_ref[...], preferred_element_type=jnp.float32) + b_ref[...]
    mu = jnp.mean(z, axis=0, keepdims=True)
    d = z - mu
    var = jnp.mean(d * d, axis=0, keepdims=True)
    h = d * lax.rsqrt(var + 1e-5) * g_ref[...] + beta_ref[...]
    gids = lax.broadcasted_iota(jnp.int32, (N, G), 1)
    onehot = (batch_ref[...] == gids).astype(jnp.float32)
    sums = lax.dot_general(onehot, h, (((0,), (0,)), ((), ())),
                           precision=lax.Precision.HIGHEST,
                           preferred_element_type=jnp.float32)
    counts = jnp.sum(onehot, axis=0)[:, None]
    hg = sums / jnp.maximum(counts, 1.0)
    o_ref[...] = jnp.dot(hg, wout_ref[...],
                         preferred_element_type=jnp.float32) + bout_ref[...]


@jax.jit
def _last_tc(e, h, s, w, b, g, beta, batch2, wout, bout):
    return pl.pallas_call(
        _last_body,
        out_shape=jax.ShapeDtypeStruct((G, wout.shape[1]), jnp.float32),
    )(e, h, s, w, b, g, beta, batch2, wout, bout)


def kernel(x, edge_index, batch, W0, b0, Ws, bs, gammas, betas, eps, W_out, b_out):
    src = edge_index[0]
    dst = edge_index[1]
    pad = E_PAD - E
    src3 = jnp.concatenate([src, jnp.zeros((pad,), jnp.int32)]).reshape(NC, NS, CH, LANES)
    dst3 = jnp.concatenate([dst, jnp.full((pad,), N_PAD - 1, jnp.int32)]).reshape(NC, NS, CH, LANES)
    batch2 = batch.reshape(N, 1)

    h = x
    for l in range(L):
        d = h.shape[1]
        zeros = jnp.zeros((ROWS_PER_TILE, d), jnp.float32)
        s = _segsum_sc(h, src3, dst3, zeros)
        e = (1.0 + eps[l]).reshape(1, 1)
        w = W0 if l == 0 else Ws[l - 1]
        b = b0 if l == 0 else bs[l - 1]
        if l < L - 1:
            h = _mid_tc(e, h, s, w, b.reshape(1, H), gammas[l].reshape(1, H),
                        betas[l].reshape(1, H))
        else:
            return _last_tc(e, h, s, w, b.reshape(1, H), gammas[l].reshape(1, H),
                            betas[l].reshape(1, H), batch2, W_out,
                            b_out.reshape(1, W_out.shape[1]))
